# baseline (speedup 1.0000x reference)
import functools

import jax
import jax.numpy as jnp
from jax.experimental import pallas as pl
from jax.experimental.pallas import tpu as pltpu
from jax.experimental.pallas import tpu_sc as plsc

B, N1, N2 = 8, 1024, 4096
DIM1, DIM2, DOUT = 256, 128, 128
N2B = 1024
NJ = N2 // N2B
NIDX = 3 * B * N2
GWIN = 128


def _top3_body(xyz1t_ref, points1_ref, xyz2p_ref, w1_ref, s1_ref, t1_ref,
               feats1_ref, idxrows_ref, wcols_ref):
    b = pl.program_id(0)
    j = pl.program_id(1)

    @pl.when(j == 0)
    def _():
        f1 = jnp.dot(points1_ref[0], w1_ref[...],
                     preferred_element_type=jnp.float32)
        feats1_ref[0] = jnp.maximum(f1 * s1_ref[0] + t1_ref[0], 0.0)

    x1t = xyz1t_ref[0]
    x2 = xyz2p_ref[0]
    n1sq = (x1t[0:1, :] * x1t[0:1, :]
            + x1t[1:2, :] * x1t[1:2, :]
            + x1t[2:3, :] * x1t[2:3, :])
    n2sq = (x2[:, 0:1] * x2[:, 0:1]
            + x2[:, 1:2] * x2[:, 1:2]
            + x2[:, 2:3] * x2[:, 2:3])
    p = jnp.dot(x2, x1t, preferred_element_type=jnp.float32)
    d = -2.0 * p + n2sq + n1sq

    iota = jax.lax.broadcasted_iota(jnp.int32, (N2B, N1), 1).astype(jnp.float32)
    big_d = jnp.float32(1e30)
    big_i = jnp.float32(2.0 ** 30)
    mins, idxs = [], []
    for _k in range(3):
        mn = jnp.min(d, axis=1, keepdims=True)
        ik = jnp.min(jnp.where(d == mn, iota, big_i), axis=1, keepdims=True)
        mins.append(mn)
        idxs.append(ik)
        d = jnp.where(iota == ik, big_d, d)

    r = [1.0 / (m + 1e-8) for m in mins]
    norm = r[0] + r[1] + r[2]

    boff = (b * N1).astype(jnp.float32)
    rows = [jnp.transpose(ik + boff, (1, 0)) for ik in idxs]
    rows.append(jnp.zeros((8 - 3, N2B), jnp.float32))
    idxrows_ref[0] = jnp.concatenate(rows, axis=0)

    cols = [rk / norm for rk in r]
    cols.append(jnp.zeros((N2B, 8 - 3), jnp.float32))
    wcols_ref[0] = jnp.concatenate(cols, axis=1)


@jax.jit
def _top3(xyz1t, points1, xyz2p, w1, s1, t1):
    return pl.pallas_call(
        _top3_body,
        grid=(B, NJ),
        in_specs=[
            pl.BlockSpec((1, 8, N1), lambda b, j: (b, 0, 0)),
            pl.BlockSpec((1, N1, DIM1), lambda b, j: (b, 0, 0)),
            pl.BlockSpec((1, N2B, 8), lambda b, j: (b, j, 0)),
            pl.BlockSpec((DIM1, DOUT), lambda b, j: (0, 0)),
            pl.BlockSpec((1, DOUT), lambda b, j: (0, 0)),
            pl.BlockSpec((1, DOUT), lambda b, j: (0, 0)),
        ],
        out_specs=[
            pl.BlockSpec((1, N1, DOUT), lambda b, j: (b, 0, 0)),
            pl.BlockSpec((1, 8, N2B), lambda b, j: (b, 0, j)),
            pl.BlockSpec((1, N2B, 8), lambda b, j: (b, j, 0)),
        ],
        out_shape=[
            jax.ShapeDtypeStruct((B, N1, DOUT), jnp.float32),
            jax.ShapeDtypeStruct((B, 8, N2), jnp.float32),
            jax.ShapeDtypeStruct((B, N2, 8), jnp.float32),
        ],
    )(xyz1t, points1, xyz2p, w1, s1, t1)


@jax.jit
def _sc_gather(feats1_flat, idx_all):
    vector_mesh = plsc.VectorSubcoreMesh(
        core_axis_name="core", subcore_axis_name="subcore")

    @pl.kernel(out_type=jax.ShapeDtypeStruct((NIDX, DOUT), jnp.float32),
               mesh=vector_mesh)
    def _gather_kernel(x_hbm, i_hbm, o_hbm):
        def body(i_vmem, o_vmem):
            pltpu.sync_copy(x_hbm.at[i_vmem.at[0]], o_vmem)

        pltpu.emit_pipeline(
            body,
            grid=(NIDX // GWIN,),
            in_specs=[pl.BlockSpec((1, GWIN), index_map=lambda i: (0, i))],
            out_specs=[pl.BlockSpec((GWIN, DOUT), index_map=lambda i: (i, 0))],
            core_axis_name=("core", "subcore"),
            dimension_semantics=(pltpu.PARALLEL,),
        )(i_hbm, o_hbm)

    return _gather_kernel(feats1_flat, idx_all)


def _combine_body(points2_ref, w2_ref, s2_ref, t2_ref, wcols_ref,
                  g0_ref, g1_ref, g2_ref, out_ref):
    f2 = jnp.dot(points2_ref[0], w2_ref[...],
                 preferred_element_type=jnp.float32)
    f2 = jnp.maximum(f2 * s2_ref[0] + t2_ref[0], 0.0)
    wc = wcols_ref[0]
    out_ref[0] = (f2
                  + wc[:, 0:1] * g0_ref[0, 0]
                  + wc[:, 1:2] * g1_ref[0, 0]
                  + wc[:, 2:3] * g2_ref[0, 0])


@jax.jit
def _combine(points2, w2, s2, t2, wcols, g):
    gspec = lambda k: pl.BlockSpec(
        (1, 1, N2B, DOUT), lambda b, j, _k=k: (_k, b, j, 0))
    return pl.pallas_call(
        _combine_body,
        grid=(B, NJ),
        in_specs=[
            pl.BlockSpec((1, N2B, DIM2), lambda b, j: (b, j, 0)),
            pl.BlockSpec((DIM2, DOUT), lambda b, j: (0, 0)),
            pl.BlockSpec((1, DOUT), lambda b, j: (0, 0)),
            pl.BlockSpec((1, DOUT), lambda b, j: (0, 0)),
            pl.BlockSpec((1, N2B, 8), lambda b, j: (b, j, 0)),
            gspec(0), gspec(1), gspec(2),
        ],
        out_specs=pl.BlockSpec((1, N2B, DOUT), lambda b, j: (b, j, 0)),
        out_shape=jax.ShapeDtypeStruct((B, N2, DOUT), jnp.float32),
    )(points2, w2, s2, t2, wcols, g, g, g)


def kernel(xyz1, points1, xyz2, points2, W1, b1, gamma1, beta1, rm1, rv1,
           W2, b2, gamma2, beta2, rm2, rv2):
    s1 = (gamma1 / jnp.sqrt(rv1 + 1e-5))[None, :]
    t1 = ((b1 - rm1) * s1[0] + beta1)[None, :]
    s2 = (gamma2 / jnp.sqrt(rv2 + 1e-5))[None, :]
    t2 = ((b2 - rm2) * s2[0] + beta2)[None, :]

    xyz2p = jnp.pad(xyz2, ((0, 0), (0, 0), (0, 5)))
    xyz1t = jnp.pad(xyz1, ((0, 0), (0, 0), (0, 5)))
    xyz1t = jnp.transpose(xyz1t, (0, 2, 1))

    feats1, idxrows, wcols = _top3(xyz1t, points1, xyz2p, W1.T, s1, t1)

    idx_all = (jnp.transpose(idxrows[:, 0:3, :], (1, 0, 2))
               .reshape(1, NIDX).astype(jnp.int32))
    gathered = _sc_gather(feats1.reshape(B * N1, DOUT), idx_all)
    g = gathered.reshape(3, B, N2, DOUT)

    return _combine(points2, W2.T, s2, t2, wcols, g)


# speedup vs baseline: 1.0844x; 1.0844x over previous
import jax
import jax.numpy as jnp
from jax.experimental import pallas as pl
from jax.experimental.pallas import tpu as pltpu
from jax.experimental.pallas import tpu_sc as plsc

B, N1, N2 = 8, 1024, 4096
DIM1, DIM2, DOUT = 256, 128, 128
N2B = 1024
NJ = N2 // N2B
NH = 2
BH = B // NH
NIDX_H = 3 * BH * N2
GWIN = 128


def _top3_body(xyz1t_ref, points1_ref, xyz2s_ref, w1_ref, s1_ref, t1_ref,
               feats1_ref, idxrows_ref, wcols_ref):
    b = pl.program_id(0)
    j = pl.program_id(1)

    @pl.when(j == 0)
    def _():
        f1 = jnp.dot(points1_ref[0], w1_ref[...],
                     preferred_element_type=jnp.float32)
        feats1_ref[0] = jnp.maximum(f1 * s1_ref[0] + t1_ref[0], 0.0)

    x1t = xyz1t_ref[0]
    x2s = xyz2s_ref[0]
    n1sq = (x1t[0:1, :] * x1t[0:1, :]
            + x1t[1:2, :] * x1t[1:2, :]
            + x1t[2:3, :] * x1t[2:3, :])
    n2sq = 0.25 * (x2s[:, 0:1] * x2s[:, 0:1]
                   + x2s[:, 1:2] * x2s[:, 1:2]
                   + x2s[:, 2:3] * x2s[:, 2:3])
    p2 = jnp.dot(x2s, x1t, preferred_element_type=jnp.float32)
    d = p2 + n2sq + n1sq

    iota = jax.lax.broadcasted_iota(jnp.int32, (N2B, N1), 1).astype(jnp.float32)
    big_d = jnp.float32(1e30)
    big_i = jnp.float32(2.0 ** 30)
    mins, idxs = [], []
    for _k in range(3):
        mn = jnp.min(d, axis=1, keepdims=True)
        ik = jnp.min(jnp.where(d == mn, iota, big_i), axis=1, keepdims=True)
        mins.append(mn)
        idxs.append(ik)
        d = jnp.where(iota == ik, big_d, d)

    r = [1.0 / (m + 1e-8) for m in mins]
    norm = r[0] + r[1] + r[2]

    boff = (b * N1).astype(jnp.float32)
    rows = [jnp.transpose(ik + boff, (1, 0)) for ik in idxs]
    rows.append(jnp.zeros((8 - 3, N2B), jnp.float32))
    idxrows_ref[0] = jnp.concatenate(rows, axis=0)

    cols = [rk / norm for rk in r]
    cols.append(jnp.zeros((N2B, 8 - 3), jnp.float32))
    wcols_ref[0] = jnp.concatenate(cols, axis=1)


def _top3(h, xyz1t, points1, xyz2s, w1, s1, t1):
    off = h * BH
    return pl.pallas_call(
        _top3_body,
        grid=(BH, NJ),
        in_specs=[
            pl.BlockSpec((1, 8, N1), lambda b, j: (b + off, 0, 0)),
            pl.BlockSpec((1, N1, DIM1), lambda b, j: (b + off, 0, 0)),
            pl.BlockSpec((1, N2B, 8), lambda b, j: (b + off, j, 0)),
            pl.BlockSpec((DIM1, DOUT), lambda b, j: (0, 0)),
            pl.BlockSpec((1, DOUT), lambda b, j: (0, 0)),
            pl.BlockSpec((1, DOUT), lambda b, j: (0, 0)),
        ],
        out_specs=[
            pl.BlockSpec((1, N1, DOUT), lambda b, j: (b, 0, 0)),
            pl.BlockSpec((1, 8, N2B), lambda b, j: (b, 0, j)),
            pl.BlockSpec((1, N2B, 8), lambda b, j: (b, j, 0)),
        ],
        out_shape=[
            jax.ShapeDtypeStruct((BH, N1, DOUT), jnp.float32),
            jax.ShapeDtypeStruct((BH, 8, N2), jnp.float32),
            jax.ShapeDtypeStruct((BH, N2, 8), jnp.float32),
        ],
    )(xyz1t, points1, xyz2s, w1, s1, t1)


def _sc_gather(feats1_flat, idx_all):
    vector_mesh = plsc.VectorSubcoreMesh(
        core_axis_name="core", subcore_axis_name="subcore")

    @pl.kernel(out_type=jax.ShapeDtypeStruct((NIDX_H, DOUT), jnp.float32),
               mesh=vector_mesh)
    def _gather_kernel(x_hbm, i_hbm, o_hbm):
        def body(i_vmem, o_vmem):
            pltpu.sync_copy(x_hbm.at[i_vmem.at[0]], o_vmem)

        pltpu.emit_pipeline(
            body,
            grid=(NIDX_H // GWIN,),
            in_specs=[pl.BlockSpec((1, GWIN), index_map=lambda i: (0, i))],
            out_specs=[pl.BlockSpec((GWIN, DOUT), index_map=lambda i: (i, 0))],
            core_axis_name=("core", "subcore"),
            dimension_semantics=(pltpu.PARALLEL,),
        )(i_hbm, o_hbm)

    return _gather_kernel(feats1_flat, idx_all)


def _combine_body(points2_ref, w2_ref, s2_ref, t2_ref, wcols_ref,
                  g0_ref, g1_ref, g2_ref, out_ref):
    f2 = jnp.dot(points2_ref[0], w2_ref[...],
                 preferred_element_type=jnp.float32)
    f2 = jnp.maximum(f2 * s2_ref[0] + t2_ref[0], 0.0)
    wc = wcols_ref[0]
    out_ref[0] = (f2
                  + wc[:, 0:1] * g0_ref[0, 0]
                  + wc[:, 1:2] * g1_ref[0, 0]
                  + wc[:, 2:3] * g2_ref[0, 0])


def _combine(h, points2, w2, s2, t2, wcols, g):
    off = h * BH
    gspec = lambda k: pl.BlockSpec(
        (1, 1, N2B, DOUT), lambda b, j, _k=k: (_k, b, j, 0))
    return pl.pallas_call(
        _combine_body,
        grid=(BH, NJ),
        in_specs=[
            pl.BlockSpec((1, N2B, DIM2), lambda b, j: (b + off, j, 0)),
            pl.BlockSpec((DIM2, DOUT), lambda b, j: (0, 0)),
            pl.BlockSpec((1, DOUT), lambda b, j: (0, 0)),
            pl.BlockSpec((1, DOUT), lambda b, j: (0, 0)),
            pl.BlockSpec((1, N2B, 8), lambda b, j: (b, j, 0)),
            gspec(0), gspec(1), gspec(2),
        ],
        out_specs=pl.BlockSpec((1, N2B, DOUT), lambda b, j: (b, j, 0)),
        out_shape=jax.ShapeDtypeStruct((BH, N2, DOUT), jnp.float32),
    )(points2, w2, s2, t2, wcols, g, g, g)


def kernel(xyz1, points1, xyz2, points2, W1, b1, gamma1, beta1, rm1, rv1,
           W2, b2, gamma2, beta2, rm2, rv2):
    s1 = (gamma1 / jnp.sqrt(rv1 + 1e-5))[None, :]
    t1 = ((b1 - rm1) * s1[0] + beta1)[None, :]
    s2 = (gamma2 / jnp.sqrt(rv2 + 1e-5))[None, :]
    t2 = ((b2 - rm2) * s2[0] + beta2)[None, :]

    xyz2s = jnp.pad(-2.0 * xyz2, ((0, 0), (0, 0), (0, 5)))
    xyz1t = jnp.pad(xyz1, ((0, 0), (0, 0), (0, 5)))
    xyz1t = jnp.transpose(xyz1t, (0, 2, 1))

    w1 = W1.T
    w2 = W2.T
    halves = []
    for h in range(NH):
        feats1, idxrows, wcols = _top3(h, xyz1t, points1, xyz2s, w1, s1, t1)
        idx_all = (jnp.transpose(idxrows[:, 0:3, :], (1, 0, 2))
                   .reshape(1, NIDX_H).astype(jnp.int32))
        gathered = _sc_gather(feats1.reshape(BH * N1, DOUT), idx_all)
        g = gathered.reshape(3, BH, N2, DOUT)
        halves.append(_combine(h, points2, w2, s2, t2, wcols, g))

    return jnp.concatenate(halves, axis=0)


# speedup vs baseline: 1.1473x; 1.0580x over previous
import jax
import jax.numpy as jnp
from jax.experimental import pallas as pl
from jax.experimental.pallas import tpu as pltpu
from jax.experimental.pallas import tpu_sc as plsc

B, N1, N2 = 8, 1024, 4096
DIM1, DIM2, DOUT = 256, 128, 128
N2B = 1024
NJ = N2 // N2B
NH = 2
BH = B // NH
NIDX_H = 3 * BH * N2
GWIN = 128


def _top3_body(xyz1_ref, points1_ref, xyz2_ref, w1_ref, s1_ref, t1_ref,
               feats1_ref, idxrows_ref, wcols_ref, x1t_ref):
    b = pl.program_id(0)
    j = pl.program_id(1)

    @pl.when(j == 0)
    def _():
        f1 = jnp.dot(points1_ref[0], w1_ref[...],
                     preferred_element_type=jnp.float32)
        feats1_ref[0] = jnp.maximum(f1 * s1_ref[0] + t1_ref[0], 0.0)
        x1t = jnp.transpose(xyz1_ref[0], (1, 0))
        n1sq = (x1t[0:1, :] * x1t[0:1, :]
                + x1t[1:2, :] * x1t[1:2, :]
                + x1t[2:3, :] * x1t[2:3, :])
        x1t_ref[...] = jnp.concatenate(
            [x1t, n1sq, jnp.zeros((4, N1), jnp.float32)], axis=0)

    x1t3 = x1t_ref[0:3, :]
    n1sq = x1t_ref[3:4, :]
    x2 = xyz2_ref[0]
    n2sq = (x2[:, 0:1] * x2[:, 0:1]
            + x2[:, 1:2] * x2[:, 1:2]
            + x2[:, 2:3] * x2[:, 2:3])
    p2 = jnp.dot(-2.0 * x2, x1t3,
                 preferred_element_type=jnp.float32)
    d = p2 + n2sq + n1sq

    iota = jax.lax.broadcasted_iota(jnp.int32, (N2B, N1), 1).astype(jnp.float32)
    big_d = jnp.float32(1e30)
    big_i = jnp.float32(2.0 ** 30)
    mins, idxs = [], []
    for _k in range(3):
        mn = jnp.min(d, axis=1, keepdims=True)
        ik = jnp.min(jnp.where(d == mn, iota, big_i), axis=1, keepdims=True)
        mins.append(mn)
        idxs.append(ik)
        d = jnp.where(iota == ik, big_d, d)

    r = [1.0 / (m + 1e-8) for m in mins]
    norm = r[0] + r[1] + r[2]

    boff = (b * N1).astype(jnp.float32)
    rows = [jnp.transpose(ik + boff, (1, 0)) for ik in idxs]
    rows.append(jnp.zeros((8 - 3, N2B), jnp.float32))
    idxrows_ref[0] = jnp.concatenate(rows, axis=0)

    cols = [rk / norm for rk in r]
    cols.append(jnp.zeros((N2B, 8 - 3), jnp.float32))
    wcols_ref[0] = jnp.concatenate(cols, axis=1)


def _top3(h, xyz1, points1, xyz2, w1, s1, t1):
    off = h * BH
    return pl.pallas_call(
        _top3_body,
        grid=(BH, NJ),
        in_specs=[
            pl.BlockSpec((1, N1, 3), lambda b, j: (b + off, 0, 0)),
            pl.BlockSpec((1, N1, DIM1), lambda b, j: (b + off, 0, 0)),
            pl.BlockSpec((1, N2B, 3), lambda b, j: (b + off, j, 0)),
            pl.BlockSpec((DIM1, DOUT), lambda b, j: (0, 0)),
            pl.BlockSpec((1, DOUT), lambda b, j: (0, 0)),
            pl.BlockSpec((1, DOUT), lambda b, j: (0, 0)),
        ],
        out_specs=[
            pl.BlockSpec((1, N1, DOUT), lambda b, j: (b, 0, 0)),
            pl.BlockSpec((1, 8, N2B), lambda b, j: (b, 0, j)),
            pl.BlockSpec((1, N2B, 8), lambda b, j: (b, j, 0)),
        ],
        out_shape=[
            jax.ShapeDtypeStruct((BH, N1, DOUT), jnp.float32),
            jax.ShapeDtypeStruct((BH, 8, N2), jnp.float32),
            jax.ShapeDtypeStruct((BH, N2, 8), jnp.float32),
        ],
        scratch_shapes=[pltpu.VMEM((8, N1), jnp.float32)],
    )(xyz1, points1, xyz2, w1, s1, t1)


def _sc_gather(feats1_flat, idx_all):
    vector_mesh = plsc.VectorSubcoreMesh(
        core_axis_name="core", subcore_axis_name="subcore")

    @pl.kernel(out_type=jax.ShapeDtypeStruct((NIDX_H, DOUT), jnp.float32),
               mesh=vector_mesh)
    def _gather_kernel(x_hbm, i_hbm, o_hbm):
        def body(i_vmem, o_vmem):
            pltpu.sync_copy(x_hbm.at[i_vmem.at[0]], o_vmem)

        pltpu.emit_pipeline(
            body,
            grid=(NIDX_H // GWIN,),
            in_specs=[pl.BlockSpec((1, GWIN), index_map=lambda i: (0, i))],
            out_specs=[pl.BlockSpec((GWIN, DOUT), index_map=lambda i: (i, 0))],
            core_axis_name=("core", "subcore"),
            dimension_semantics=(pltpu.PARALLEL,),
        )(i_hbm, o_hbm)

    return _gather_kernel(feats1_flat, idx_all)


def _combine_body(points2_ref, w2_ref, s2_ref, t2_ref, wcols_ref,
                  g0_ref, g1_ref, g2_ref, outbuf_ref, out_ref):
    del outbuf_ref
    f2 = jnp.dot(points2_ref[0], w2_ref[...],
                 preferred_element_type=jnp.float32)
    f2 = jnp.maximum(f2 * s2_ref[0] + t2_ref[0], 0.0)
    wc = wcols_ref[0]
    out_ref[0] = (f2
                  + wc[:, 0:1] * g0_ref[0, 0]
                  + wc[:, 1:2] * g1_ref[0, 0]
                  + wc[:, 2:3] * g2_ref[0, 0])


def _combine(h, points2, w2, s2, t2, wcols, g, outbuf):
    off = h * BH
    gspec = lambda k: pl.BlockSpec(
        (1, 1, N2B, DOUT), lambda b, j, _k=k: (_k, b, j, 0))
    return pl.pallas_call(
        _combine_body,
        grid=(BH, NJ),
        in_specs=[
            pl.BlockSpec((1, N2B, DIM2), lambda b, j: (b + off, j, 0)),
            pl.BlockSpec((DIM2, DOUT), lambda b, j: (0, 0)),
            pl.BlockSpec((1, DOUT), lambda b, j: (0, 0)),
            pl.BlockSpec((1, DOUT), lambda b, j: (0, 0)),
            pl.BlockSpec((1, N2B, 8), lambda b, j: (b, j, 0)),
            gspec(0), gspec(1), gspec(2),
            pl.BlockSpec(memory_space=pltpu.MemorySpace.HBM),
        ],
        out_specs=pl.BlockSpec((1, N2B, DOUT), lambda b, j: (b + off, j, 0)),
        out_shape=jax.ShapeDtypeStruct((B, N2, DOUT), jnp.float32),
        input_output_aliases={8: 0},
    )(points2, w2, s2, t2, wcols, g, g, g, outbuf)


def kernel(xyz1, points1, xyz2, points2, W1, b1, gamma1, beta1, rm1, rv1,
           W2, b2, gamma2, beta2, rm2, rv2):
    s1 = (gamma1 / jnp.sqrt(rv1 + 1e-5))[None, :]
    t1 = ((b1 - rm1) * s1[0] + beta1)[None, :]
    s2 = (gamma2 / jnp.sqrt(rv2 + 1e-5))[None, :]
    t2 = ((b2 - rm2) * s2[0] + beta2)[None, :]

    w1 = W1.T
    w2 = W2.T
    out = jnp.zeros((B, N2, DOUT), jnp.float32)
    for h in range(NH):
        feats1, idxrows, wcols = _top3(h, xyz1, points1, xyz2, w1, s1, t1)
        idx_all = (jnp.transpose(idxrows[:, 0:3, :], (1, 0, 2))
                   .reshape(1, NIDX_H).astype(jnp.int32))
        gathered = _sc_gather(feats1.reshape(BH * N1, DOUT), idx_all)
        g = gathered.reshape(3, BH, N2, DOUT)
        out = _combine(h, points2, w2, s2, t2, wcols, g, out)

    return out


# speedup vs baseline: 1.1948x; 1.0414x over previous
import functools

import jax
import jax.numpy as jnp
from jax.experimental import pallas as pl
from jax.experimental.pallas import tpu as pltpu
from jax.experimental.pallas import tpu_sc as plsc

B, N1, N2 = 8, 1024, 4096
DIM1, DIM2, DOUT = 256, 128, 128
N2B = 1024
NJ = N2 // N2B
NH = 2
BH = B // NH
NIDX_H = 3 * BH * N2
GWIN = 128


def _top3_body(has_dummy, xyz1t_ref, points1_ref, xyz2t_ref,
               w1_ref, s1_ref, t1_ref, feats1_ref, idxrows_ref, wcols_ref,
               *rest):
    if has_dummy:
        dummy_ref, n1sq_ref = rest
    else:
        (n1sq_ref,) = rest
    b = pl.program_id(0)
    j = pl.program_id(1)

    x1t3 = xyz1t_ref[0]

    @pl.when(j == 0)
    def _():
        f1 = jnp.dot(points1_ref[0], w1_ref[...],
                     preferred_element_type=jnp.float32)
        feats1_ref[0] = jnp.maximum(f1 * s1_ref[0] + t1_ref[0], 0.0)
        n1sq_ref[0:1, :] = (x1t3[0:1, :] * x1t3[0:1, :]
                            + x1t3[1:2, :] * x1t3[1:2, :]
                            + x1t3[2:3, :] * x1t3[2:3, :])

    if has_dummy:
        @pl.when(jnp.logical_and(b == 0, j == 0))
        def _():
            dummy_ref[0] = jnp.zeros((N2B, DOUT), jnp.float32)

    n1sq = n1sq_ref[0:1, :]
    x2t = xyz2t_ref[0]
    n2row = (x2t[0:1, :] * x2t[0:1, :]
             + x2t[1:2, :] * x2t[1:2, :]
             + x2t[2:3, :] * x2t[2:3, :])
    n2sq = jnp.transpose(n2row, (1, 0))
    p2 = jax.lax.dot_general(
        -2.0 * x2t, x1t3, (((0,), (0,)), ((), ())),
        preferred_element_type=jnp.float32)
    d = p2 + n2sq + n1sq

    iota = jax.lax.broadcasted_iota(jnp.int32, (N2B, N1), 1).astype(jnp.float32)
    big_d = jnp.float32(1e30)
    big_i = jnp.float32(2.0 ** 30)
    mins, idxs = [], []
    for _k in range(3):
        mn = jnp.min(d, axis=1, keepdims=True)
        ik = jnp.min(jnp.where(d == mn, iota, big_i), axis=1, keepdims=True)
        mins.append(mn)
        idxs.append(ik)
        d = jnp.where(iota == ik, big_d, d)

    r = [1.0 / (m + 1e-8) for m in mins]
    norm = r[0] + r[1] + r[2]

    boff = (b * N1).astype(jnp.float32)
    rows = [jnp.transpose(ik + boff, (1, 0)) for ik in idxs]
    rows.append(jnp.zeros((8 - 3, N2B), jnp.float32))
    idxrows_ref[0] = jnp.concatenate(rows, axis=0)

    cols = [rk / norm for rk in r]
    cols.append(jnp.zeros((N2B, 8 - 3), jnp.float32))
    wcols_ref[0] = jnp.concatenate(cols, axis=1)


def _top3(h, xyz1t, points1, xyz2t, w1, s1, t1):
    off = h * BH
    has_dummy = h == 0
    out_specs = [
        pl.BlockSpec((1, N1, DOUT), lambda b, j: (b, 0, 0)),
        pl.BlockSpec((1, 8, N2B), lambda b, j: (b, 0, j)),
        pl.BlockSpec((1, N2B, 8), lambda b, j: (b, j, 0)),
    ]
    out_shape = [
        jax.ShapeDtypeStruct((BH, N1, DOUT), jnp.float32),
        jax.ShapeDtypeStruct((BH, 8, N2), jnp.float32),
        jax.ShapeDtypeStruct((BH, N2, 8), jnp.float32),
    ]
    if has_dummy:
        out_specs.append(pl.BlockSpec((1, N2B, DOUT), lambda b, j: (0, 0, 0)))
        out_shape.append(jax.ShapeDtypeStruct((B, N2, DOUT), jnp.float32))
    return pl.pallas_call(
        functools.partial(_top3_body, has_dummy),
        grid=(BH, NJ),
        in_specs=[
            pl.BlockSpec((1, 3, N1), lambda b, j: (b + off, 0, 0)),
            pl.BlockSpec((1, N1, DIM1), lambda b, j: (b + off, 0, 0)),
            pl.BlockSpec((1, 3, N2B), lambda b, j: (b + off, 0, j)),
            pl.BlockSpec((DIM1, DOUT), lambda b, j: (0, 0)),
            pl.BlockSpec((1, DOUT), lambda b, j: (0, 0)),
            pl.BlockSpec((1, DOUT), lambda b, j: (0, 0)),
        ],
        out_specs=out_specs,
        out_shape=out_shape,
        scratch_shapes=[pltpu.VMEM((8, N1), jnp.float32)],
    )(xyz1t, points1, xyz2t, w1, s1, t1)


def _sc_gather(feats1_flat, idx_all):
    vector_mesh = plsc.VectorSubcoreMesh(
        core_axis_name="core", subcore_axis_name="subcore")

    @pl.kernel(out_type=jax.ShapeDtypeStruct((NIDX_H, DOUT), jnp.float32),
               mesh=vector_mesh)
    def _gather_kernel(x_hbm, i_hbm, o_hbm):
        def body(i_vmem, o_vmem):
            pltpu.sync_copy(x_hbm.at[i_vmem.at[0]], o_vmem)

        pltpu.emit_pipeline(
            body,
            grid=(NIDX_H // GWIN,),
            in_specs=[pl.BlockSpec((1, GWIN), index_map=lambda i: (0, i))],
            out_specs=[pl.BlockSpec((GWIN, DOUT), index_map=lambda i: (i, 0))],
            core_axis_name=("core", "subcore"),
            dimension_semantics=(pltpu.PARALLEL,),
        )(i_hbm, o_hbm)

    return _gather_kernel(feats1_flat, idx_all)


def _combine_body(points2_ref, w2_ref, s2_ref, t2_ref, wcols_ref,
                  g0_ref, g1_ref, g2_ref, outbuf_ref, out_ref):
    del outbuf_ref
    f2 = jnp.dot(points2_ref[0], w2_ref[...],
                 preferred_element_type=jnp.float32)
    f2 = jnp.maximum(f2 * s2_ref[0] + t2_ref[0], 0.0)
    wc = wcols_ref[0]
    out_ref[0] = (f2
                  + wc[:, 0:1] * g0_ref[0, 0]
                  + wc[:, 1:2] * g1_ref[0, 0]
                  + wc[:, 2:3] * g2_ref[0, 0])


def _combine(h, points2, w2, s2, t2, wcols, g, outbuf):
    off = h * BH
    gspec = lambda k: pl.BlockSpec(
        (1, 1, N2B, DOUT), lambda b, j, _k=k: (_k, b, j, 0))
    return pl.pallas_call(
        _combine_body,
        grid=(BH, NJ),
        in_specs=[
            pl.BlockSpec((1, N2B, DIM2), lambda b, j: (b + off, j, 0)),
            pl.BlockSpec((DIM2, DOUT), lambda b, j: (0, 0)),
            pl.BlockSpec((1, DOUT), lambda b, j: (0, 0)),
            pl.BlockSpec((1, DOUT), lambda b, j: (0, 0)),
            pl.BlockSpec((1, N2B, 8), lambda b, j: (b, j, 0)),
            gspec(0), gspec(1), gspec(2),
            pl.BlockSpec(memory_space=pltpu.MemorySpace.HBM),
        ],
        out_specs=pl.BlockSpec((1, N2B, DOUT), lambda b, j: (b + off, j, 0)),
        out_shape=jax.ShapeDtypeStruct((B, N2, DOUT), jnp.float32),
        input_output_aliases={8: 0},
    )(points2, w2, s2, t2, wcols, g, g, g, outbuf)


def kernel(xyz1, points1, xyz2, points2, W1, b1, gamma1, beta1, rm1, rv1,
           W2, b2, gamma2, beta2, rm2, rv2):
    s1 = (gamma1 / jnp.sqrt(rv1 + 1e-5))[None, :]
    t1 = ((b1 - rm1) * s1[0] + beta1)[None, :]
    s2 = (gamma2 / jnp.sqrt(rv2 + 1e-5))[None, :]
    t2 = ((b2 - rm2) * s2[0] + beta2)[None, :]

    w1 = W1.T
    w2 = W2.T
    xyz1t = jnp.transpose(xyz1, (0, 2, 1))
    xyz2t = jnp.transpose(xyz2, (0, 2, 1))

    out = None
    for h in range(NH):
        res = _top3(h, xyz1t, points1, xyz2t, w1, s1, t1)
        if h == 0:
            feats1, idxrows, wcols, out = res
        else:
            feats1, idxrows, wcols = res
        idx_all = (jnp.transpose(idxrows[:, 0:3, :], (1, 0, 2))
                   .reshape(1, NIDX_H).astype(jnp.int32))
        gathered = _sc_gather(feats1.reshape(BH * N1, DOUT), idx_all)
        g = gathered.reshape(3, BH, N2, DOUT)
        out = _combine(h, points2, w2, s2, t2, wcols, g, out)

    return out
